# Initial kernel scaffold; baseline (speedup 1.0000x reference)
import jax
import jax.numpy as jnp
from jax.experimental import pallas as pl


def kernel(x_code, x_size, edge_index, edge_type, size_emb, code_emb, W0, root0, b0, W1, root1, b1, W2, root2, b2, W3, root3, b3, lin_W, lin_b):
    raise NotImplementedError("write your pallas kernel here")


# speedup vs baseline: 3.2609x; 3.2609x over previous
import functools

import jax
import jax.numpy as jnp
from jax import lax
from jax.experimental import pallas as pl
from jax.experimental.pallas import tpu as pltpu
from jax.experimental.pallas import tpu_sc as plsc

N = 100000
E = 1600000
R = 3
NW = 32
BLK = 128
EPW = 50048
EPAD = EPW * NW
NBLK = EPW // BLK
RNPAD = 301056
RNROW = RNPAD // 16
CROWS = 392
TPT = RNPAD // 16
CCH = 2352
NACC = 100096
APT = NACC // 16
TCB = 1000
GRID = N // TCB
ZROWS = 368

_mesh = plsc.VectorSubcoreMesh(core_axis_name="c", subcore_axis_name="s")
_sc_params = pltpu.CompilerParams(use_tc_tiling_on_sc=False,
                                  needs_layout_passes=False)


def _wid():
  return lax.axis_index("c") * 16 + lax.axis_index("s")


_BCAST_DNUMS = lax.GatherDimensionNumbers(
    offset_dims=(), collapsed_slice_dims=(0,), start_index_map=(0,))


def _bcast_lane(v16, t):
  return lax.gather(v16, jnp.full((16, 1), t, jnp.int32),
                    dimension_numbers=_BCAST_DNUMS, slice_sizes=(1,),
                    mode=lax.GatherScatterMode.PROMISE_IN_BOUNDS)


def _scale_rows(rows, wbuf):
  for j in range(BLK // 16):
    w16 = wbuf[pl.ds(j * 16, 16)]
    for t in range(16):
      wb = _bcast_lane(w16, t)
      e = j * 16 + t
      rows[e, :] = rows[e, :] * wb


HSZ = 8192


def _dedup_scatter_add(src, key_ref, acc_at, tbl, abuf, sidx, dump):
  for j in range(BLK // 16):
    abuf[pl.ds(j * 16, 16)] = jnp.ones((16,), jnp.int32)

  def round_body(r, cnt):
    del r, cnt
    for j in range(BLK // 16):
      sl = pl.ds(j * 16, 16)
      d = key_ref[sl]
      act = abuf[sl] > 0
      tags = lax.iota(jnp.int32, 16) + (j * 16)
      plsc.store_scatter(tbl, [d & (HSZ - 1)], tags, mask=act)
    newcnt = jnp.int32(0)
    for j in range(BLK // 16):
      sl = pl.ds(j * 16, 16)
      d = key_ref[sl]
      act = abuf[sl] > 0
      tags = lax.iota(jnp.int32, 16) + (j * 16)
      back = plsc.load_gather(tbl, [d & (HSZ - 1)])
      win = jnp.logical_and(act, back == tags)
      sidx[sl] = jnp.where(win, d, dump)
      rem = jnp.logical_and(act, jnp.logical_not(win)).astype(jnp.int32)
      abuf[sl] = rem
      newcnt = newcnt + jnp.sum(rem)
    pltpu.sync_copy(src, acc_at(sidx), add=True)
    return newcnt

  lax.fori_loop(0, 4, round_body, jnp.int32(BLK))


@functools.partial(
    pl.kernel,
    out_type=(
        jax.ShapeDtypeStruct((EPAD,), jnp.int32),
        jax.ShapeDtypeStruct((EPAD,), jnp.int32),
        jax.ShapeDtypeStruct((2, RNROW, 16), jnp.float32),
    ),
    mesh=_mesh,
    scratch_types=[
        pltpu.VMEM((BLK,), jnp.int32),
        pltpu.VMEM((BLK,), jnp.int32),
        pltpu.VMEM((BLK,), jnp.int32),
        pltpu.VMEM((BLK,), jnp.int32),
        pltpu.VMEM((BLK,), jnp.int32),
        pltpu.VMEM((BLK, 16), jnp.float32),
        pltpu.VMEM((CROWS, 16), jnp.float32),
        pltpu.VMEM((HSZ,), jnp.int32),
        pltpu.VMEM((BLK,), jnp.int32),
        pltpu.VMEM((BLK,), jnp.int32),
        pltpu.VMEM_SHARED((RNROW, 16), jnp.float32),
        pltpu.SemaphoreType.DMA,
    ],
    compiler_params=_sc_params,
)
def _prep_edges(src_hbm, dst_hbm, et_hbm, g_hbm, s_hbm, cnt_hbm,
                sbuf, dbuf, ebuf, gbuf, qbuf, hot, ztile, tbl, abuf, sidx,
                cnt_sh, sem):
  c = lax.axis_index("c")
  s_id = lax.axis_index("s")
  wid = _wid()
  rpt = RNROW // 16

  def zinit(i, _):
    ztile[i, :] = jnp.zeros((16,), jnp.float32)
    return 0
  lax.fori_loop(0, CROWS, zinit, 0)

  def zcopy(k, _):
    pltpu.sync_copy(ztile, cnt_sh.at[pl.ds(s_id * rpt + k * CROWS, CROWS)])
    return 0
  lax.fori_loop(0, rpt // CROWS, zcopy, 0)
  plsc.subcore_barrier()

  def body(i, _):
    base = wid * EPW + i * BLK
    pltpu.sync_copy(src_hbm.at[pl.ds(base, BLK)], sbuf)
    pltpu.sync_copy(dst_hbm.at[pl.ds(base, BLK)], dbuf)
    pltpu.sync_copy(et_hbm.at[pl.ds(base, BLK)], ebuf)
    for j in range(BLK // 16):
      sl = pl.ds(j * 16, 16)
      et = ebuf[sl]
      gbuf[sl] = jnp.minimum(et, R - 1) * N + sbuf[sl]
      qbuf[sl] = et * N + dbuf[sl]
    pltpu.sync_copy(gbuf, g_hbm.at[pl.ds(base, BLK)])
    pltpu.sync_copy(qbuf, s_hbm.at[pl.ds(base, BLK)])
    for j in range(BLK // 16):
      sl = pl.ds(j * 16, 16)
      q = qbuf[sl]
      sbuf[sl] = q >> 4
      lo = q & 15
      for t in range(16):
        hot[j * 16 + t, :] = (lax.iota(jnp.int32, 16) == _bcast_lane(lo, t)
                              ).astype(jnp.float32)
    _dedup_scatter_add(hot, sbuf, lambda s: cnt_sh.at[s], tbl, abuf, sidx,
                       RNROW - 1)
    return 0

  lax.fori_loop(0, NBLK, body, 0)
  plsc.subcore_barrier()

  def ccopy(k, _):
    off = s_id * rpt + k * CROWS
    pltpu.sync_copy(cnt_sh.at[pl.ds(off, CROWS)], ztile)
    pltpu.sync_copy(ztile, cnt_hbm.at[c, pl.ds(off, CROWS)])
    return 0
  lax.fori_loop(0, rpt // CROWS, ccopy, 0)


@functools.partial(
    pl.kernel,
    out_type=jax.ShapeDtypeStruct((EPAD,), jnp.float32),
    mesh=_mesh,
    scratch_types=[
        pltpu.VMEM((BLK,), jnp.int32),
        pltpu.VMEM((BLK,), jnp.float32),
        pltpu.VMEM((CCH,), jnp.float32),
        pltpu.VMEM_SHARED((RNPAD,), jnp.float32),
        pltpu.SemaphoreType.DMA,
        pltpu.SemaphoreType.DMA,
    ],
    compiler_params=_sc_params,
)
def _prep_w(s_hbm, wtab_hbm, w_hbm, qbuf, wbuf, chunk, wtab_sh, sem, sem2):
  s_id = lax.axis_index("s")
  wid = _wid()

  def stage(k, _):
    off = s_id * TPT + k * CCH
    pltpu.sync_copy(wtab_hbm.at[pl.ds(off, CCH)], chunk)
    pltpu.sync_copy(chunk, wtab_sh.at[pl.ds(off, CCH)])
    return 0
  lax.fori_loop(0, TPT // CCH, stage, 0)
  plsc.subcore_barrier()

  def body(i, _):
    base = wid * EPW + i * BLK
    pltpu.sync_copy(s_hbm.at[pl.ds(base, BLK)], qbuf)
    pltpu.async_copy(wtab_sh.at[qbuf], wbuf, sem2).wait()
    pltpu.sync_copy(wbuf, w_hbm.at[pl.ds(base, BLK)])
    return 0

  lax.fori_loop(0, NBLK, body, 0)


@functools.partial(
    pl.kernel,
    out_type=jax.ShapeDtypeStruct((2, NACC, 16), jnp.float32),
    mesh=_mesh,
    scratch_types=[
        pltpu.VMEM((BLK,), jnp.int32),
        pltpu.VMEM((BLK,), jnp.int32),
        pltpu.VMEM((BLK,), jnp.float32),
        pltpu.VMEM((BLK, 16), jnp.float32),
        pltpu.VMEM((ZROWS, 16), jnp.float32),
        pltpu.VMEM((HSZ,), jnp.int32),
        pltpu.VMEM((BLK,), jnp.int32),
        pltpu.VMEM((BLK,), jnp.int32),
        pltpu.VMEM_SHARED((NACC, 16), jnp.float32),
        pltpu.SemaphoreType.DMA,
    ],
    compiler_params=_sc_params,
)
def _agg(tab_hbm, g_hbm, dst_hbm, w_hbm, out_hbm,
         gbuf, dbuf, wbuf, rows, ztile, tbl, abuf, sidx, acc_sh, sem):
  c = lax.axis_index("c")
  s_id = lax.axis_index("s")
  wid = _wid()

  def zinit(i, _):
    ztile[i, :] = jnp.zeros((16,), jnp.float32)
    return 0
  lax.fori_loop(0, ZROWS, zinit, 0)

  def zcopy(k, _):
    pltpu.sync_copy(ztile, acc_sh.at[pl.ds(s_id * APT + k * ZROWS, ZROWS)])
    return 0
  lax.fori_loop(0, APT // ZROWS, zcopy, 0)
  plsc.subcore_barrier()

  def body(i, _):
    base = wid * EPW + i * BLK
    pltpu.sync_copy(g_hbm.at[pl.ds(base, BLK)], gbuf)
    pltpu.sync_copy(dst_hbm.at[pl.ds(base, BLK)], dbuf)
    pltpu.sync_copy(w_hbm.at[pl.ds(base, BLK)], wbuf)
    pltpu.async_copy(tab_hbm.at[gbuf], rows, sem).wait()
    _scale_rows(rows, wbuf)
    _dedup_scatter_add(rows, dbuf, lambda s: acc_sh.at[s], tbl, abuf, sidx,
                       NACC - 1)
    return 0

  lax.fori_loop(0, NBLK, body, 0)
  plsc.subcore_barrier()

  def ocopy(k, _):
    off = s_id * APT + k * ZROWS
    pltpu.sync_copy(acc_sh.at[pl.ds(off, ZROWS)], ztile)
    pltpu.sync_copy(ztile, out_hbm.at[c, pl.ds(off, ZROWS)])
    return 0
  lax.fori_loop(0, APT // ZROWS, ocopy, 0)


def _wtab_body(cnt_ref, out_ref):
  i = pl.program_id(0)
  csum = cnt_ref[0] + cnt_ref[1]
  flat = (i * 1024 + lax.broadcasted_iota(jnp.int32, (8, 128), 0) * 128
          + lax.broadcasted_iota(jnp.int32, (8, 128), 1))
  w = 1.0 / jnp.maximum(csum, 1.0)
  out_ref[...] = jnp.where(flat < R * N, w, 0.0)


def _build_wtab(cnt):
  cnt3 = cnt.reshape(2, RNPAD // 128, 128)
  out = pl.pallas_call(
      _wtab_body,
      grid=(RNPAD // 1024,),
      in_specs=[pl.BlockSpec((2, 8, 128), lambda i: (0, i, 0))],
      out_specs=pl.BlockSpec((8, 128), lambda i: (i, 0)),
      out_shape=jax.ShapeDtypeStruct((RNPAD // 128, 128), jnp.float32),
  )(cnt3)
  return out.reshape(RNPAD)


def _emb_body(se_ref, ce_ref, w_ref, root_ref, b_ref,
              xs_ref, xc_ref, ta_ref, tb_ref, root_out):
  xs = xs_ref[0, 0, :]
  xc = xc_ref[0, 0, :]
  oh_s = (xs[:, None] == lax.broadcasted_iota(jnp.int32, (TCB, 16), 1)
          ).astype(jnp.float32)
  oh_c = (xc[:, None] == lax.broadcasted_iota(jnp.int32, (TCB, 256), 1)
          ).astype(jnp.float32)
  xa = jax.lax.dot(oh_s, se_ref[...], preferred_element_type=jnp.float32,
                precision=lax.Precision.HIGHEST)
  xb = jax.lax.dot(oh_c, ce_ref[...], preferred_element_type=jnp.float32,
                precision=lax.Precision.HIGHEST)
  x = jnp.concatenate([xa, xb], axis=1)
  for r in range(R):
    y = jax.lax.dot(x, w_ref[r], preferred_element_type=jnp.float32,
                precision=lax.Precision.HIGHEST)
    ta_ref[r] = y[:, :16]
    tb_ref[r] = y[:, 16:]
  root_out[...] = (jax.lax.dot(x, root_ref[...],
                               preferred_element_type=jnp.float32,
                precision=lax.Precision.HIGHEST)
                   + b_ref[...])


def _layer0_tables(xs3, xc3, se_pad, ce_pad, w_pad, root, b):
  return pl.pallas_call(
      _emb_body,
      grid=(GRID,),
      in_specs=[
          pl.BlockSpec((16, 4), lambda i: (0, 0)),
          pl.BlockSpec((256, 32), lambda i: (0, 0)),
          pl.BlockSpec((R, 36, 32), lambda i: (0, 0, 0)),
          pl.BlockSpec((36, 24), lambda i: (0, 0)),
          pl.BlockSpec((1, 24), lambda i: (0, 0)),
          pl.BlockSpec((1, 1, TCB), lambda i: (i, 0, 0)),
          pl.BlockSpec((1, 1, TCB), lambda i: (i, 0, 0)),
      ],
      out_specs=[
          pl.BlockSpec((R, TCB, 16), lambda i: (0, i, 0)),
          pl.BlockSpec((R, TCB, 16), lambda i: (0, i, 0)),
          pl.BlockSpec((TCB, 24), lambda i: (i, 0)),
      ],
      out_shape=[
          jax.ShapeDtypeStruct((R, N, 16), jnp.float32),
          jax.ShapeDtypeStruct((R, N, 16), jnp.float32),
          jax.ShapeDtypeStruct((N, 24), jnp.float32),
      ],
  )(se_pad, ce_pad, w_pad, root, b, xs3, xc3)


def _mid_body(din, dout, w_ref, root_ref, b_ref, root_in, pa_ref, pb_ref,
              tab_ref, root_out):
  agg = pa_ref[0] + pa_ref[1]
  if pb_ref is not None:
    aggb = pb_ref[0] + pb_ref[1]
    agg = jnp.concatenate([agg, aggb[:, :din - 16]], axis=1)
  else:
    agg = agg[:, :din]
  x = jax.nn.relu(root_in[...] + agg)
  for r in range(R):
    tab_ref[r] = jax.lax.dot(x, w_ref[r], preferred_element_type=jnp.float32,
                precision=lax.Precision.HIGHEST)
  root_out[...] = (jax.lax.dot(x, root_ref[...],
                               preferred_element_type=jnp.float32,
                precision=lax.Precision.HIGHEST)
                   + b_ref[...])


def _mid_layer(din, dout, w_pad, root, b, root_prev, pa, pb):
  has_b = pb is not None
  body = lambda *refs: _mid_body(
      din, dout, refs[0], refs[1], refs[2], refs[3], refs[4],
      refs[5] if has_b else None, refs[-2], refs[-1])
  in_specs = [
      pl.BlockSpec((R, din, 16), lambda i: (0, 0, 0)),
      pl.BlockSpec((din, dout), lambda i: (0, 0)),
      pl.BlockSpec((1, dout), lambda i: (0, 0)),
      pl.BlockSpec((TCB, din), lambda i: (i, 0)),
      pl.BlockSpec((2, TCB, 16), lambda i: (0, i, 0)),
  ]
  args = [w_pad, root, b, root_prev, pa]
  if has_b:
    in_specs.append(pl.BlockSpec((2, TCB, 16), lambda i: (0, i, 0)))
    args.append(pb)
  return pl.pallas_call(
      body,
      grid=(GRID,),
      in_specs=in_specs,
      out_specs=[
          pl.BlockSpec((R, TCB, 16), lambda i: (0, i, 0)),
          pl.BlockSpec((TCB, dout), lambda i: (i, 0)),
      ],
      out_shape=[
          jax.ShapeDtypeStruct((R, N, 16), jnp.float32),
          jax.ShapeDtypeStruct((N, dout), jnp.float32),
      ],
  )(*args)


def _final_body(lw_ref, lb_ref, root_in, p_ref, out_ref):
  agg = (p_ref[0] + p_ref[1])[:, :4]
  x = jax.nn.relu(root_in[...] + agg)
  out_ref[...] = (jax.lax.dot(x, lw_ref[...],
                              preferred_element_type=jnp.float32,
                precision=lax.Precision.HIGHEST)
                  + lb_ref[...])


def _final_layer(lin_W, lin_b, root_prev, p):
  return pl.pallas_call(
      _final_body,
      grid=(GRID,),
      in_specs=[
          pl.BlockSpec((4, 2), lambda i: (0, 0)),
          pl.BlockSpec((1, 2), lambda i: (0, 0)),
          pl.BlockSpec((TCB, 4), lambda i: (i, 0)),
          pl.BlockSpec((2, TCB, 16), lambda i: (0, i, 0)),
      ],
      out_specs=pl.BlockSpec((TCB, 2), lambda i: (i, 0)),
      out_shape=jax.ShapeDtypeStruct((N, 2), jnp.float32),
  )(lin_W, lin_b, root_prev, p)


def _pad_w(W, din, dpad):
  out = jnp.zeros((R, din, dpad), jnp.float32)
  return out.at[:, :W.shape[1], :W.shape[2]].set(W)


def kernel(x_code, x_size, edge_index, edge_type, size_emb, code_emb,
           W0, root0, b0, W1, root1, b1, W2, root2, b2, W3, root3, b3,
           lin_W, lin_b):
  src = edge_index[0].astype(jnp.int32)
  dst = edge_index[1].astype(jnp.int32)
  et = edge_type.astype(jnp.int32)

  pad = EPAD - E
  src_p = jnp.concatenate([src, jnp.zeros((pad,), jnp.int32)])
  dst_p = jnp.concatenate([dst, jnp.zeros((pad,), jnp.int32)])
  et_p = jnp.concatenate([et, jnp.full((pad,), R, jnp.int32)])

  g_idx, s_idx, cnt = _prep_edges(src_p, dst_p, et_p)
  wtab = _build_wtab(cnt.reshape(2, RNPAD))
  w_edge = _prep_w(s_idx, wtab)

  xs3 = x_size.astype(jnp.int32).reshape(GRID, 1, TCB)
  xc3 = x_code.astype(jnp.int32).reshape(GRID, 1, TCB)
  se_pad = jnp.zeros((16, 4), jnp.float32).at[:15].set(size_emb)
  ce_pad = jnp.zeros((256, 32), jnp.float32).at[:202].set(code_emb)

  w0_pad = _pad_w(W0, 36, 32)
  t0a, t0b, root_acc = _layer0_tables(xs3, xc3, se_pad, ce_pad,
                                      w0_pad, root0, b0.reshape(1, -1))

  pa = _agg(t0a.reshape(R * N, 16), g_idx, dst_p, w_edge)
  pb = _agg(t0b.reshape(R * N, 16), g_idx, dst_p, w_edge)

  dims = [(24, 16, W1, root1, b1), (16, 16, W2, root2, b2),
          (8, 16, W3, root3, b3)]
  douts = [16, 8, 4]
  for li, (din, dpad, W, root, b) in enumerate(dims):
    w_pad = _pad_w(W, din, dpad)
    tab, root_acc = _mid_layer(din, douts[li], w_pad, root,
                               b.reshape(1, -1), root_acc, pa, pb)
    pb = None
    pa = _agg(tab.reshape(R * N, 16), g_idx, dst_p, w_edge)

  return _final_layer(lin_W, lin_b.reshape(1, -1), root_acc, pa)
